# baseline (speedup 1.0000x reference)
import functools

import jax
import jax.numpy as jnp
from jax import lax
from jax.experimental import pallas as pl
from jax.experimental.pallas import tpu as pltpu
from jax.experimental.pallas import tpu_sc as plsc

_TOL = 0.01
_MAXITER = 16

_N = 10000
_E = 320000
_CP = 48
_CE = 2048
_CPH = 80
_EPAD = 2 * _CPH * _CE
_NB = 4


def _projT_body(w_ref, nf_ref, b_ref, o_ref):
    o_ref[...] = (
        lax.dot_general(w_ref[...], nf_ref[...], (((1,), (1,)), ((), ())),
                        preferred_element_type=jnp.float32)
        + b_ref[...]
    )


def _project_T(nf, WpT, bp):
    N, D = nf.shape
    return pl.pallas_call(
        _projT_body,
        out_shape=jax.ShapeDtypeStruct((_CP, N), jnp.float32),
    )(WpT, nf, bp.reshape(_CP, 1))


@functools.partial(
    pl.kernel,
    out_type=jax.ShapeDtypeStruct((2, _CP, _N), jnp.float32),
    mesh=plsc.VectorSubcoreMesh(core_axis_name="c", subcore_axis_name="s"),
    compiler_params=pltpu.CompilerParams(
        use_tc_tiling_on_sc=False, needs_layout_passes=False),
    scratch_types=(
        [pltpu.VMEM((_N,), jnp.float32)] * 6
        + [pltpu.VMEM((2, _CE), jnp.int32)] * _NB
        + [pltpu.VMEM((_CE,), jnp.float32)] * _NB
        + [pltpu.SemaphoreType.DMA] * _NB
    ),
)
def _sc_matvec(vT_hbm, ech_hbm, adjh_hbm, zeros_hbm, av_hbm, *scr):
    vp = scr[0:3]
    ap = scr[3:6]
    ebuf = scr[6:6 + _NB]
    abuf = scr[6 + _NB:6 + 2 * _NB]
    esem = scr[6 + 2 * _NB:6 + 3 * _NB]
    cid = lax.axis_index("c")
    sid = lax.axis_index("s")
    cbase = cid * _CPH
    for k in range(3):
        pltpu.sync_copy(vT_hbm.at[3 * sid + k], vp[k])
        pltpu.sync_copy(zeros_hbm, ap[k])
    for q in range(_NB - 1):
        pltpu.async_copy(ech_hbm.at[cbase + q], ebuf[q], esem[q])
        pltpu.async_copy(adjh_hbm.at[cbase + q], abuf[q], esem[q])

    def outer_body(o, carry):
        for b in range(_NB):
            i = o * _NB + b
            pltpu.make_async_copy(ech_hbm.at[cbase], ebuf[b], esem[b]).wait()
            pltpu.make_async_copy(adjh_hbm.at[cbase], abuf[b], esem[b]).wait()

            nxt = (b + _NB - 1) % _NB

            @pl.when(i + _NB - 1 < _CPH)
            def _():
                pltpu.async_copy(ech_hbm.at[cbase + i + _NB - 1],
                                 ebuf[nxt], esem[nxt])
                pltpu.async_copy(adjh_hbm.at[cbase + i + _NB - 1],
                                 abuf[nxt], esem[nxt])

            def vec_body(j, c2):
                colv = ebuf[b][0, pl.ds(j * 16, 16)]
                rowv = ebuf[b][1, pl.ds(j * 16, 16)]
                adjv = abuf[b][pl.ds(j * 16, 16)]
                for k in range(3):
                    g = plsc.load_gather(vp[k], [colv])
                    plsc.addupdate_scatter(ap[k], [rowv], g * adjv)
                return c2

            lax.fori_loop(0, _CE // 16, vec_body, 0)
        return carry

    lax.fori_loop(0, _CPH // _NB, outer_body, 0)
    for k in range(3):
        pltpu.sync_copy(ap[k], av_hbm.at[cid, 3 * sid + k])


_NIDP = 1024


@functools.partial(
    pl.kernel,
    out_type=jax.ShapeDtypeStruct((_NIDP, _CP), jnp.float32),
    mesh=plsc.VectorSubcoreMesh(core_axis_name="c", subcore_axis_name="s"),
    compiler_params=pltpu.CompilerParams(
        use_tc_tiling_on_sc=False, needs_layout_passes=False),
    scratch_types=[
        pltpu.VMEM((32,), jnp.int32),
        pltpu.VMEM((32, _CP), jnp.float32),
        pltpu.SemaphoreType.DMA,
    ],
)
def _sc_ids_gather(sol_hbm, ids_hbm, out_hbm, ids_v, rows_v, sem):
    w = lax.axis_index("c") * 16 + lax.axis_index("s")
    pltpu.sync_copy(ids_hbm.at[w], ids_v)
    pltpu.async_copy(sol_hbm.at[ids_v], rows_v, sem).wait()
    pltpu.sync_copy(rows_v, out_hbm.at[pl.ds(w * 32, 32)])


def kernel(node_features, adj_values, e0, W, b, edge_index, ids):
    D, C = W.shape
    WpT = jnp.zeros((_CP, D), jnp.float32).at[:C, :].set(W.T)
    bp = jnp.zeros((_CP,), jnp.float32).at[:C].set(b)
    xT = _project_T(node_features, WpT, bp)

    pad = _EPAD - _E
    colp = jnp.pad(edge_index[1], (0, pad)).reshape(2 * _CPH, _CE)
    rowp = jnp.pad(edge_index[0], (0, pad)).reshape(2 * _CPH, _CE)
    ech = jnp.stack([colp, rowp], axis=1)
    adjh = jnp.pad(adj_values, (0, pad)).reshape(2 * _CPH, _CE)
    zeros = jnp.zeros((_N,), jnp.float32)

    epsilon = jax.nn.sigmoid(e0)
    c = 1.0 - epsilon

    def matvec(v):
        av2 = _sc_matvec(v, ech, adjh, zeros)
        return v - c * (av2[0] + av2[1])

    sol, _ = jax.scipy.sparse.linalg.cg(matvec, xT, tol=_TOL, maxiter=_MAXITER)

    ids_p = jnp.pad(ids, (0, _NIDP - ids.shape[0])).reshape(32, 32)
    outp = _sc_ids_gather(sol.T, ids_p)
    return outp[: ids.shape[0], :C]
